# Initial kernel scaffold; baseline (speedup 1.0000x reference)
import functools

import jax
import jax.numpy as jnp
from jax import lax
from jax.experimental import pallas as pl
from jax.experimental.pallas import tpu as pltpu
from jax.experimental.pallas import tpu_sc as plsc

N = 10000
E = 320000
D = 128
H = 64
G = 64

NC, NS = 2, 16
NW = NC * NS
EPT = E // NW
CH = 125
NCH = EPT // CH
NPAD = 10240
RPT = NPAD // NS

_mesh = plsc.VectorSubcoreMesh(core_axis_name="c", subcore_axis_name="s")
_f32 = jnp.float32


def _wid_sid_cid():
    cid = lax.axis_index("c")
    sid = lax.axis_index("s")
    return cid * NS + sid, sid, cid


@functools.partial(
    pl.kernel,
    out_type=jax.ShapeDtypeStruct((NC, NPAD), _f32),
    mesh=_mesh,
    scratch_types=[
        pltpu.VMEM((NCH, CH), jnp.int32),
        pltpu.VMEM((128,), _f32),
        pltpu.VMEM_SHARED((NPAD,), _f32),
    ],
)
def _deg_kernel(dst_hbm, zn_hbm, out_hbm, idxd_v, ones_v, deg_sh):
    wid, sid, cid = _wid_sid_cid()
    pltpu.sync_copy(zn_hbm, deg_sh.at[pl.ds(sid * RPT, RPT)])
    pltpu.sync_copy(dst_hbm.at[wid], idxd_v)
    for k in range(8):
        ones_v[pl.ds(k * 16, 16)] = jnp.ones((16,), _f32)
    plsc.subcore_barrier()

    def body(j, carry):
        pltpu.sync_copy(ones_v.at[pl.ds(0, CH)],
                        deg_sh.at[idxd_v.at[j]], add=True)
        return carry

    lax.fori_loop(0, NCH, body, 0)
    plsc.subcore_barrier()
    pltpu.sync_copy(deg_sh.at[pl.ds(sid * RPT, RPT)],
                    out_hbm.at[cid, pl.ds(sid * RPT, RPT)])


@functools.partial(
    pl.kernel,
    out_type=jax.ShapeDtypeStruct((NC, NPAD, H), _f32),
    mesh=_mesh,
    scratch_types=[
        pltpu.VMEM((NCH, CH), jnp.int32),
        pltpu.VMEM((NCH, CH), jnp.int32),
        pltpu.VMEM((CH, H), _f32),
        pltpu.VMEM((CH, H), _f32),
        pltpu.VMEM_SHARED((NPAD, H), _f32),
        pltpu.SemaphoreType.DMA,
        pltpu.SemaphoreType.DMA,
    ],
)
def _agg_kernel(hs_hbm, src_hbm, dst_hbm, zrows_hbm, out_hbm,
                idxs_v, idxd_v, rows0, rows1, acc_sh, sem0, sem1):
    wid, sid, cid = _wid_sid_cid()
    pltpu.sync_copy(zrows_hbm, acc_sh.at[pl.ds(sid * RPT, RPT)])
    pltpu.sync_copy(src_hbm.at[wid], idxs_v)
    pltpu.sync_copy(dst_hbm.at[wid], idxd_v)
    plsc.subcore_barrier()

    bufs = (rows0, rows1)
    sems = (sem0, sem1)
    pltpu.async_copy(hs_hbm.at[idxs_v.at[0]], rows0, sem0)
    pltpu.async_copy(hs_hbm.at[idxs_v.at[1]], rows1, sem1)
    dummy = hs_hbm.at[pl.ds(0, CH)]

    def outer(g, carry):
        for b in range(2):
            j = g * 2 + b
            pltpu.make_async_copy(dummy, bufs[b], sems[b]).wait()
            pltpu.sync_copy(bufs[b], acc_sh.at[idxd_v.at[j]], add=True)

            @pl.when(j + 2 < NCH)
            def _issue():
                pltpu.async_copy(hs_hbm.at[idxs_v.at[j + 2]], bufs[b], sems[b])
        return carry

    lax.fori_loop(0, NCH // 2, outer, 0)
    plsc.subcore_barrier()
    pltpu.sync_copy(acc_sh.at[pl.ds(sid * RPT, RPT)],
                    out_hbm.at[cid, pl.ds(sid * RPT, RPT)])


def _mm1_body(degp_ref, x_ref, w1_ref, hs1_ref, dinv_ref):
    deg = degp_ref[0, :] + degp_ref[1, :] + 1.0
    dinv = lax.rsqrt(deg)
    dinv_ref[...] = dinv
    h = jnp.dot(x_ref[...], w1_ref[...], preferred_element_type=_f32)
    hs1_ref[...] = h * dinv[:, None]


def _mm2_body(accp_ref, hs1_ref, dinv_ref, w2_ref, b1_ref, hs2_ref):
    dinv = dinv_ref[...]
    tot = (accp_ref[0] + accp_ref[1] + hs1_ref[...]) * dinv[:, None]
    h1 = jnp.maximum(tot + b1_ref[...][None, :], 0.0)
    hs2_ref[...] = jnp.dot(h1, w2_ref[...], preferred_element_type=_f32) * dinv[:, None]


def _fin_body(accp_ref, hs2_ref, dinv_ref, b2_ref, batch_ref, wf_ref, bf_ref,
              out_ref):
    dinv = dinv_ref[...]
    tot = (accp_ref[0] + accp_ref[1] + hs2_ref[...]) * dinv[:, None]
    h2 = jnp.maximum(tot + b2_ref[...][None, :], 0.0)
    b = batch_ref[...]
    onehot_t = (b[None, :] == lax.broadcasted_iota(jnp.int32, (G, NPAD), 0))
    onehot_t = onehot_t.astype(_f32)
    sums = jnp.dot(onehot_t, h2, preferred_element_type=_f32)
    counts = jnp.sum(onehot_t, axis=1)
    pooled = sums / jnp.maximum(counts, 1.0)[:, None]
    z = jnp.dot(pooled, wf_ref[...], preferred_element_type=_f32) + bf_ref[...][None, :]
    out_ref[...] = 1.0 / (1.0 + jnp.exp(-z))


def kernel(x, edge_index, batch, W1, b1, W2, b2, Wf, bf):
    src3d = edge_index[0].reshape(NW, NCH, CH)
    dst3d = edge_index[1].reshape(NW, NCH, CH)
    x_pad = jnp.pad(x, ((0, NPAD - N), (0, 0)))
    batch_pad = jnp.concatenate([batch, jnp.full((NPAD - N,), G, jnp.int32)])
    zn = jnp.zeros((RPT,), _f32)
    zrows = jnp.zeros((RPT, H), _f32)

    degp = _deg_kernel(dst3d, zn)
    hs1, dinv = pl.pallas_call(
        _mm1_body,
        out_shape=[jax.ShapeDtypeStruct((NPAD, H), _f32),
                   jax.ShapeDtypeStruct((NPAD,), _f32)],
    )(degp, x_pad, W1)
    acc1 = _agg_kernel(hs1, src3d, dst3d, zrows)
    hs2 = pl.pallas_call(
        _mm2_body,
        out_shape=jax.ShapeDtypeStruct((NPAD, H), _f32),
    )(acc1, hs1, dinv, W2, b1)
    acc2 = _agg_kernel(hs2, src3d, dst3d, zrows)
    out = pl.pallas_call(
        _fin_body,
        out_shape=jax.ShapeDtypeStruct((G, 1), _f32),
    )(acc2, hs2, dinv, b2, batch_pad, Wf, bf)
    return out


# speedup vs baseline: 41.5245x; 41.5245x over previous
import functools

import jax
import jax.numpy as jnp
from jax import lax
from jax.experimental import pallas as pl
from jax.experimental.pallas import tpu as pltpu
from jax.experimental.pallas import tpu_sc as plsc

N = 10000
E = 320000
D = 128
H = 64
G = 64

NC, NS = 2, 16
NW = NC * NS
EPT = E // NW
CH = 125
NCH = EPT // CH
NPAD = 10240
RPT = NPAD // NS

_mesh = plsc.VectorSubcoreMesh(core_axis_name="c", subcore_axis_name="s")
_f32 = jnp.float32


def _wid_sid_cid():
    cid = lax.axis_index("c")
    sid = lax.axis_index("s")
    return cid * NS + sid, sid, cid


@functools.partial(
    pl.kernel,
    out_type=jax.ShapeDtypeStruct((NC, NPAD), _f32),
    mesh=_mesh,
    scratch_types=[
        pltpu.VMEM((NCH, CH), jnp.int32),
        pltpu.VMEM((128,), _f32),
        pltpu.VMEM_SHARED((NPAD,), _f32),
    ],
)
def _deg_kernel(dst_hbm, zn_hbm, out_hbm, idxd_v, ones_v, deg_sh):
    wid, sid, cid = _wid_sid_cid()
    pltpu.sync_copy(zn_hbm, deg_sh.at[pl.ds(sid * RPT, RPT)])
    pltpu.sync_copy(dst_hbm.at[wid], idxd_v)
    for k in range(8):
        ones_v[pl.ds(k * 16, 16)] = jnp.ones((16,), _f32)
    plsc.subcore_barrier()

    def body(j, carry):
        pltpu.sync_copy(ones_v.at[pl.ds(0, CH)],
                        deg_sh.at[idxd_v.at[j]], add=True)
        return carry

    lax.fori_loop(0, NCH, body, 0)
    plsc.subcore_barrier()
    pltpu.sync_copy(deg_sh.at[pl.ds(sid * RPT, RPT)],
                    out_hbm.at[cid, pl.ds(sid * RPT, RPT)])


@functools.partial(
    pl.kernel,
    out_type=jax.ShapeDtypeStruct((NC, NPAD, H), _f32),
    mesh=_mesh,
    compiler_params=pltpu.CompilerParams(use_tc_tiling_on_sc=False),
    scratch_types=[
        pltpu.VMEM((NCH, CH), jnp.int32),
        pltpu.VMEM((NCH, CH), jnp.int32),
        pltpu.VMEM((CH, H), _f32),
        pltpu.VMEM((CH, H), _f32),
        pltpu.VMEM_SHARED((NPAD, H), _f32),
        pltpu.SemaphoreType.DMA,
        pltpu.SemaphoreType.DMA,
    ],
)
def _agg_kernel(hs_hbm, src_hbm, dst_hbm, zrows_hbm, out_hbm,
                idxs_v, idxd_v, rows0, rows1, acc_sh, sem0, sem1):
    wid, sid, cid = _wid_sid_cid()
    pltpu.sync_copy(zrows_hbm, acc_sh.at[pl.ds(sid * RPT, RPT)])
    pltpu.sync_copy(src_hbm.at[wid], idxs_v)
    pltpu.sync_copy(dst_hbm.at[wid], idxd_v)
    plsc.subcore_barrier()

    bufs = (rows0, rows1)
    sems = (sem0, sem1)
    pltpu.async_copy(hs_hbm.at[idxs_v.at[0]], rows0, sem0)
    pltpu.async_copy(hs_hbm.at[idxs_v.at[1]], rows1, sem1)

    def outer(g, carry):
        for b in range(2):
            j = g * 2 + b
            pltpu.make_async_copy(hs_hbm.at[idxs_v.at[j]],
                                  bufs[b], sems[b]).wait()
            pltpu.sync_copy(bufs[b], acc_sh.at[idxd_v.at[j]], add=True)

            @pl.when(j + 2 < NCH)
            def _issue():
                pltpu.async_copy(hs_hbm.at[idxs_v.at[j + 2]], bufs[b], sems[b])
        return carry

    lax.fori_loop(0, NCH // 2, outer, 0)
    plsc.subcore_barrier()
    pltpu.sync_copy(acc_sh.at[pl.ds(sid * RPT, RPT)],
                    out_hbm.at[cid, pl.ds(sid * RPT, RPT)])


def _mm1_body(degp_ref, x_ref, w1_ref, hs1_ref, dinv_ref):
    deg = degp_ref[0, :] + degp_ref[1, :] + 1.0
    dinv = lax.rsqrt(deg)
    dinv_ref[...] = dinv
    h = jnp.dot(x_ref[...], w1_ref[...], preferred_element_type=_f32)
    hs1_ref[...] = h * dinv[:, None]


def _mm2_body(accp_ref, hs1_ref, dinv_ref, w2_ref, b1_ref, hs2_ref):
    dinv = dinv_ref[...]
    tot = (accp_ref[0] + accp_ref[1] + hs1_ref[...]) * dinv[:, None]
    h1 = jnp.maximum(tot + b1_ref[...][None, :], 0.0)
    hs2_ref[...] = jnp.dot(h1, w2_ref[...], preferred_element_type=_f32) * dinv[:, None]


def _fin_body(accp_ref, hs2_ref, dinv_ref, b2_ref, batch_ref, wf_ref, bf_ref,
              out_ref):
    dinv = dinv_ref[...]
    tot = (accp_ref[0] + accp_ref[1] + hs2_ref[...]) * dinv[:, None]
    h2 = jnp.maximum(tot + b2_ref[...][None, :], 0.0)
    b = batch_ref[...]
    onehot_t = (b[None, :] == lax.broadcasted_iota(jnp.int32, (G, NPAD), 0))
    onehot_t = onehot_t.astype(_f32)
    sums = jnp.dot(onehot_t, h2, preferred_element_type=_f32)
    counts = jnp.sum(onehot_t, axis=1)
    pooled = sums / jnp.maximum(counts, 1.0)[:, None]
    z = jnp.dot(pooled, wf_ref[...], preferred_element_type=_f32) + bf_ref[...][None, :]
    out_ref[...] = 1.0 / (1.0 + jnp.exp(-z))


def kernel(x, edge_index, batch, W1, b1, W2, b2, Wf, bf):
    src3d = edge_index[0].reshape(NW, NCH, CH)
    dst3d = edge_index[1].reshape(NW, NCH, CH)
    x_pad = jnp.pad(x, ((0, NPAD - N), (0, 0)))
    batch_pad = jnp.concatenate([batch, jnp.full((NPAD - N,), G, jnp.int32)])
    zn = jnp.zeros((RPT,), _f32)
    zrows = jnp.zeros((RPT, H), _f32)

    degp = _deg_kernel(dst3d, zn)
    hs1, dinv = pl.pallas_call(
        _mm1_body,
        out_shape=[jax.ShapeDtypeStruct((NPAD, H), _f32),
                   jax.ShapeDtypeStruct((NPAD,), _f32)],
    )(degp, x_pad, W1)
    acc1 = _agg_kernel(hs1, src3d, dst3d, zrows)
    hs2 = pl.pallas_call(
        _mm2_body,
        out_shape=jax.ShapeDtypeStruct((NPAD, H), _f32),
    )(acc1, hs1, dinv, W2, b1)
    acc2 = _agg_kernel(hs2, src3d, dst3d, zrows)
    out = pl.pallas_call(
        _fin_body,
        out_shape=jax.ShapeDtypeStruct((G, 1), _f32),
    )(acc2, hs2, dinv, b2, batch_pad, Wf, bf)
    return out


# speedup vs baseline: 43.8821x; 1.0568x over previous
import functools

import jax
import jax.numpy as jnp
from jax import lax
from jax.experimental import pallas as pl
from jax.experimental.pallas import tpu as pltpu
from jax.experimental.pallas import tpu_sc as plsc

N = 10000
E = 320000
D = 128
H = 64
G = 64

NC, NS = 2, 16
NW = NC * NS
EPT = E // NW
CH = 125
NCH = EPT // CH
NPAD = 10240
RPT = NPAD // NS

_mesh = plsc.VectorSubcoreMesh(core_axis_name="c", subcore_axis_name="s")
_f32 = jnp.float32


def _wid_sid_cid():
    cid = lax.axis_index("c")
    sid = lax.axis_index("s")
    return cid * NS + sid, sid, cid


@functools.partial(
    pl.kernel,
    out_type=jax.ShapeDtypeStruct((NC, NPAD), _f32),
    mesh=_mesh,
    scratch_types=[
        pltpu.VMEM((NCH, CH), jnp.int32),
        pltpu.VMEM((128,), _f32),
        pltpu.VMEM_SHARED((NPAD,), _f32),
        pltpu.SemaphoreType.DMA,
    ],
)
def _deg_kernel(dst_hbm, zn_hbm, out_hbm, idxd_v, ones_v, deg_sh, deg_sem):
    wid, sid, cid = _wid_sid_cid()
    pltpu.sync_copy(zn_hbm, deg_sh.at[pl.ds(sid * RPT, RPT)])
    pltpu.sync_copy(dst_hbm.at[wid], idxd_v)
    for k in range(8):
        ones_v[pl.ds(k * 16, 16)] = jnp.ones((16,), _f32)
    plsc.subcore_barrier()

    def body(j, carry):
        pltpu.async_copy(ones_v.at[pl.ds(0, CH)],
                         deg_sh.at[idxd_v.at[j]], deg_sem, add=True)
        return carry

    lax.fori_loop(0, NCH, body, 0)

    def drain(j, carry):
        pltpu.make_async_copy(ones_v.at[pl.ds(0, CH)],
                              deg_sh.at[idxd_v.at[j]], deg_sem).wait()
        return carry

    lax.fori_loop(0, NCH, drain, 0)
    plsc.subcore_barrier()
    pltpu.sync_copy(deg_sh.at[pl.ds(sid * RPT, RPT)],
                    out_hbm.at[cid, pl.ds(sid * RPT, RPT)])


@functools.partial(
    pl.kernel,
    out_type=jax.ShapeDtypeStruct((NC, NPAD, H), _f32),
    mesh=_mesh,
    compiler_params=pltpu.CompilerParams(use_tc_tiling_on_sc=False),
    scratch_types=[
        pltpu.VMEM((NCH, CH), jnp.int32),
        pltpu.VMEM((NCH, CH), jnp.int32),
        [pltpu.VMEM((CH, H), _f32)] * 4,
        pltpu.VMEM_SHARED((NPAD, H), _f32),
        [pltpu.SemaphoreType.DMA] * 4,
        [pltpu.SemaphoreType.DMA] * 4,
    ],
)
def _agg_kernel(hs_hbm, src_hbm, dst_hbm, zrows_hbm, out_hbm,
                idxs_v, idxd_v, bufs, acc_sh, gsems, ssems):
    wid, sid, cid = _wid_sid_cid()
    pltpu.sync_copy(zrows_hbm, acc_sh.at[pl.ds(sid * RPT, RPT)])
    pltpu.sync_copy(src_hbm.at[wid], idxs_v)
    pltpu.sync_copy(dst_hbm.at[wid], idxd_v)
    plsc.subcore_barrier()

    pltpu.async_copy(hs_hbm.at[idxs_v.at[0]], bufs[0], gsems[0])
    pltpu.async_copy(hs_hbm.at[idxs_v.at[1]], bufs[1], gsems[1])

    def outer(g, carry):
        for b in range(4):
            j = g * 4 + b
            b2 = (b + 2) % 4
            pltpu.make_async_copy(hs_hbm.at[idxs_v.at[j]],
                                  bufs[b], gsems[b]).wait()
            pltpu.async_copy(bufs[b], acc_sh.at[idxd_v.at[j]], ssems[b],
                             add=True)

            @pl.when(j >= 2)
            def _wait_sc():
                pltpu.make_async_copy(bufs[b2], acc_sh.at[idxd_v.at[j - 2]],
                                      ssems[b2]).wait()

            @pl.when(j + 2 < NCH)
            def _issue():
                pltpu.async_copy(hs_hbm.at[idxs_v.at[j + 2]],
                                 bufs[b2], gsems[b2])
        return carry

    lax.fori_loop(0, NCH // 4, outer, 0)
    pltpu.make_async_copy(bufs[2], acc_sh.at[idxd_v.at[NCH - 2]],
                          ssems[2]).wait()
    pltpu.make_async_copy(bufs[3], acc_sh.at[idxd_v.at[NCH - 1]],
                          ssems[3]).wait()
    plsc.subcore_barrier()
    pltpu.sync_copy(acc_sh.at[pl.ds(sid * RPT, RPT)],
                    out_hbm.at[cid, pl.ds(sid * RPT, RPT)])


def _mm1_body(degp_ref, x_ref, w1_ref, hs1_ref, dinv_ref):
    deg = degp_ref[0, :] + degp_ref[1, :] + 1.0
    dinv = lax.rsqrt(deg)
    dinv_ref[...] = dinv
    h = jnp.dot(x_ref[...], w1_ref[...], preferred_element_type=_f32)
    hs1_ref[...] = h * dinv[:, None]


def _mm2_body(accp_ref, hs1_ref, dinv_ref, w2_ref, b1_ref, hs2_ref):
    dinv = dinv_ref[...]
    tot = (accp_ref[0] + accp_ref[1] + hs1_ref[...]) * dinv[:, None]
    h1 = jnp.maximum(tot + b1_ref[...][None, :], 0.0)
    hs2_ref[...] = jnp.dot(h1, w2_ref[...], preferred_element_type=_f32) * dinv[:, None]


def _fin_body(accp_ref, hs2_ref, dinv_ref, b2_ref, batch_ref, wf_ref, bf_ref,
              out_ref):
    dinv = dinv_ref[...]
    tot = (accp_ref[0] + accp_ref[1] + hs2_ref[...]) * dinv[:, None]
    h2 = jnp.maximum(tot + b2_ref[...][None, :], 0.0)
    b = batch_ref[...]
    onehot_t = (b[None, :] == lax.broadcasted_iota(jnp.int32, (G, NPAD), 0))
    onehot_t = onehot_t.astype(_f32)
    sums = jnp.dot(onehot_t, h2, preferred_element_type=_f32)
    counts = jnp.sum(onehot_t, axis=1)
    pooled = sums / jnp.maximum(counts, 1.0)[:, None]
    z = jnp.dot(pooled, wf_ref[...], preferred_element_type=_f32) + bf_ref[...][None, :]
    out_ref[...] = 1.0 / (1.0 + jnp.exp(-z))


def kernel(x, edge_index, batch, W1, b1, W2, b2, Wf, bf):
    src3d = edge_index[0].reshape(NW, NCH, CH)
    dst3d = edge_index[1].reshape(NW, NCH, CH)
    x_pad = jnp.pad(x, ((0, NPAD - N), (0, 0)))
    batch_pad = jnp.concatenate([batch, jnp.full((NPAD - N,), G, jnp.int32)])
    zn = jnp.zeros((RPT,), _f32)
    zrows = jnp.zeros((RPT, H), _f32)

    degp = _deg_kernel(dst3d, zn)
    hs1, dinv = pl.pallas_call(
        _mm1_body,
        out_shape=[jax.ShapeDtypeStruct((NPAD, H), _f32),
                   jax.ShapeDtypeStruct((NPAD,), _f32)],
    )(degp, x_pad, W1)
    acc1 = _agg_kernel(hs1, src3d, dst3d, zrows)
    hs2 = pl.pallas_call(
        _mm2_body,
        out_shape=jax.ShapeDtypeStruct((NPAD, H), _f32),
    )(acc1, hs1, dinv, W2, b1)
    acc2 = _agg_kernel(hs2, src3d, dst3d, zrows)
    out = pl.pallas_call(
        _fin_body,
        out_shape=jax.ShapeDtypeStruct((G, 1), _f32),
    )(acc2, hs2, dinv, b2, batch_pad, Wf, bf)
    return out
